# baseline (device time: 419064 ns/iter reference)
import jax
import jax.numpy as jnp
from jax import lax
from jax.experimental import pallas as pl
from jax.experimental.pallas import tpu as pltpu

P = 4
D = 2048
M = 8192
NB = 4
R = 2048 // NB
BLK = 2048
HC = D // 2


def kernel(x, W1, W2):
    def body(x_hbm, w1_hbm, w2_hbm, out_hbm,
             w1b, w2b, xb, ptmp, ptmp2, sbuf, red, rs_rbuf, ag_rbuf, obuf,
             ld_sem, st_sem,
             rs_send, rs_recv, ag_send, ag_recv):
        my = lax.axis_index("i")
        left = lax.rem(my + P - 1, P)
        right = lax.rem(my + 1, P)

        def C(h):
            return pl.ds(h * HC, HC)

        def rs_rdma(slot, h):
            return pltpu.make_async_remote_copy(
                src_ref=sbuf.at[slot, :, C(h)],
                dst_ref=rs_rbuf.at[slot, :, C(h)],
                send_sem=rs_send.at[slot, h], recv_sem=rs_recv.at[slot, h],
                device_id=(right,), device_id_type=pl.DeviceIdType.MESH)

        def ag_rdma(slot, h, src):
            return pltpu.make_async_remote_copy(
                src_ref=src.at[:, C(h)],
                dst_ref=ag_rbuf.at[slot, :, C(h)],
                send_sem=ag_send.at[slot, h], recv_sem=ag_recv.at[slot, h],
                device_id=(left,), device_id_type=pl.DeviceIdType.MESH)

        barrier = pltpu.get_barrier_semaphore()
        for nbr in (left, right):
            pl.semaphore_signal(barrier, inc=1, device_id=(nbr,),
                                device_id_type=pl.DeviceIdType.MESH)
        pl.semaphore_wait(barrier, 2)

        for c in range(D // R):
            cp = pltpu.make_async_copy(w1_hbm.at[pl.ds(c * R, R), :], xb, ld_sem)
            cp.start()
            cp.wait()
            w1b[pl.ds(c * R, R), :] = xb[...].astype(jnp.bfloat16)
            cp = pltpu.make_async_copy(w2_hbm.at[pl.ds(c * R, R), :], xb, ld_sem)
            cp.start()
            cp.wait()
            w2b[pl.ds(c * R, R), :] = xb[...].astype(jnp.bfloat16)

        def load_x(b, r):
            cp = pltpu.make_async_copy(
                x_hbm.at[pl.ds(b * BLK + r * R, R), :], xb, ld_sem)
            cp.start()
            cp.wait()

        def gemm1(r, b, dst):
            load_x(b, r)
            dst[...] = jnp.dot(xb[...].astype(jnp.bfloat16), w1b[...],
                               preferred_element_type=jnp.float32
                               ).astype(jnp.bfloat16)

        def gemm2_store(h_blk, b, q):
            obuf[...] = jnp.dot(h_blk, w2b[...],
                                preferred_element_type=jnp.float32)
            cp = pltpu.make_async_copy(
                obuf, out_hbm.at[pl.ds(b * BLK + q * R, R), :], st_sem)
            cp.start()
            cp.wait()

        load_x(lax.rem(my + P - 1, P), 0)
        sbuf[0, :, :] = jnp.dot(xb[...].astype(jnp.bfloat16), w1b[...],
                                preferred_element_type=jnp.float32
                                ).astype(jnp.bfloat16)
        for h in range(2):
            rs_rdma(0, h).start()

        def tick_round(r, carry):
            rs_on = r < NB
            ag_on = r >= 1
            for s in range(P - 1):
                @pl.when(rs_on)
                def _():
                    gemm1(r, lax.rem(my + 2 * P - s - 2, P), ptmp)

                if s == P - 2:
                    @pl.when(r + 1 < NB)
                    def _():
                        gemm1(r + 1, lax.rem(my + P - 1, P), ptmp2)

                @pl.when(ag_on)
                def _():
                    if s == 0:
                        gemm2_store(red[...], my, r - 1)
                    else:
                        gemm2_store(ag_rbuf[s - 1], lax.rem(my + s, P), r - 1)

                for h in range(2):
                    @pl.when(rs_on)
                    def _():
                        rs_rdma(s, h).wait()
                        if s < P - 2:
                            sbuf[s + 1, :, C(h)] = (
                                rs_rbuf[s, :, C(h)] + ptmp[:, C(h)])
                            rs_rdma(s + 1, h).start()
                        else:
                            red[:, C(h)] = (
                                rs_rbuf[s, :, C(h)] + ptmp[:, C(h)])
                            ag_rdma(0, h, red).start()

                    if s == P - 2:
                        @pl.when(r + 1 < NB)
                        def _():
                            sbuf[0, :, C(h)] = ptmp2[:, C(h)]
                            rs_rdma(0, h).start()

                    @pl.when(ag_on)
                    def _():
                        ag_src = red if s == 0 else ag_rbuf.at[s - 1]
                        ag_rdma(s, h, ag_src).wait()
                        if s < P - 2:
                            ag_rdma(s + 1, h, ag_rbuf.at[s]).start()

                if s == 0:
                    @pl.when(r >= 2)
                    def _():
                        gemm2_store(ag_rbuf[P - 2], lax.rem(my + P - 1, P),
                                    r - 2)
            return carry

        lax.fori_loop(0, NB + 1, tick_round, 0)

        gemm2_store(ag_rbuf[P - 2], lax.rem(my + P - 1, P), NB - 1)

    return pl.pallas_call(
        body,
        out_shape=jax.ShapeDtypeStruct((M, D), jnp.float32),
        in_specs=[
            pl.BlockSpec(memory_space=pl.ANY),
            pl.BlockSpec(memory_space=pl.ANY),
            pl.BlockSpec(memory_space=pl.ANY),
        ],
        out_specs=pl.BlockSpec(memory_space=pl.ANY),
        scratch_shapes=[
            pltpu.VMEM((D, D), jnp.bfloat16),
            pltpu.VMEM((D, D), jnp.bfloat16),
            pltpu.VMEM((R, D), jnp.float32),
            pltpu.VMEM((R, D), jnp.bfloat16),
            pltpu.VMEM((R, D), jnp.bfloat16),
            pltpu.VMEM((P - 1, R, D), jnp.bfloat16),
            pltpu.VMEM((R, D), jnp.bfloat16),
            pltpu.VMEM((P - 1, R, D), jnp.bfloat16),
            pltpu.VMEM((P - 1, R, D), jnp.bfloat16),
            pltpu.VMEM((R, D), jnp.float32),
            pltpu.SemaphoreType.DMA,
            pltpu.SemaphoreType.DMA,
            pltpu.SemaphoreType.DMA((P - 1, 2)),
            pltpu.SemaphoreType.DMA((P - 1, 2)),
            pltpu.SemaphoreType.DMA((P - 1, 2)),
            pltpu.SemaphoreType.DMA((P - 1, 2)),
        ],
        compiler_params=pltpu.CompilerParams(
            collective_id=0, vmem_limit_bytes=64 * 1024 * 1024),
    )(x, W1, W2)


# device time: 408103 ns/iter; 1.0269x vs baseline; 1.0269x over previous
import jax
import jax.numpy as jnp
from jax import lax
from jax.experimental import pallas as pl
from jax.experimental.pallas import tpu as pltpu

P = 4
D = 2048
M = 8192
NB = 8
R = 2048 // NB
BLK = 2048
HR = R // 2


def kernel(x, W1, W2):
    def body(x_hbm, w1_hbm, w2_hbm, out_hbm,
             w1b, w2b, xb, ptmp, ptmp2, sbuf, red, rs_rbuf, ag_rbuf, obuf,
             ld_sem, st_sem,
             rs_send, rs_recv, ag_send, ag_recv):
        my = lax.axis_index("i")
        left = lax.rem(my + P - 1, P)
        right = lax.rem(my + 1, P)

        def C(h):
            return pl.ds(h * HR, HR)

        def rs_rdma(slot, h):
            return pltpu.make_async_remote_copy(
                src_ref=sbuf.at[slot, C(h), :],
                dst_ref=rs_rbuf.at[slot, C(h), :],
                send_sem=rs_send.at[slot, h], recv_sem=rs_recv.at[slot, h],
                device_id=(right,), device_id_type=pl.DeviceIdType.MESH)

        def ag_rdma(slot, h, src):
            return pltpu.make_async_remote_copy(
                src_ref=src.at[C(h), :],
                dst_ref=ag_rbuf.at[slot, C(h), :],
                send_sem=ag_send.at[slot, h], recv_sem=ag_recv.at[slot, h],
                device_id=(left,), device_id_type=pl.DeviceIdType.MESH)

        barrier = pltpu.get_barrier_semaphore()
        for nbr in (left, right):
            pl.semaphore_signal(barrier, inc=1, device_id=(nbr,),
                                device_id_type=pl.DeviceIdType.MESH)
        pl.semaphore_wait(barrier, 2)

        for c in range(D // R):
            cp = pltpu.make_async_copy(w1_hbm.at[pl.ds(c * R, R), :], xb, ld_sem)
            cp.start()
            cp.wait()
            w1b[pl.ds(c * R, R), :] = xb[...].astype(jnp.bfloat16)
            cp = pltpu.make_async_copy(w2_hbm.at[pl.ds(c * R, R), :], xb, ld_sem)
            cp.start()
            cp.wait()
            w2b[pl.ds(c * R, R), :] = xb[...].astype(jnp.bfloat16)

        def load_x(b, r):
            cp = pltpu.make_async_copy(
                x_hbm.at[pl.ds(b * BLK + r * R, R), :], xb, ld_sem)
            cp.start()
            cp.wait()

        def gemm1(r, b, dst):
            load_x(b, r)
            dst[...] = jnp.dot(xb[...].astype(jnp.bfloat16), w1b[...],
                               preferred_element_type=jnp.float32
                               ).astype(jnp.bfloat16)

        def gemm2_store(h_blk, b, q):
            obuf[...] = jnp.dot(h_blk, w2b[...],
                                preferred_element_type=jnp.float32)
            cp = pltpu.make_async_copy(
                obuf, out_hbm.at[pl.ds(b * BLK + q * R, R), :], st_sem)
            cp.start()
            cp.wait()

        load_x(lax.rem(my + P - 1, P), 0)
        sbuf[0, :, :] = jnp.dot(xb[...].astype(jnp.bfloat16), w1b[...],
                                preferred_element_type=jnp.float32
                                ).astype(jnp.bfloat16)
        for h in range(2):
            rs_rdma(0, h).start()

        def tick_round(r, carry):
            rs_on = r < NB
            ag_on = r >= 1
            for s in range(P - 1):
                @pl.when(rs_on)
                def _():
                    gemm1(r, lax.rem(my + 2 * P - s - 2, P), ptmp)

                if s == P - 2:
                    @pl.when(r + 1 < NB)
                    def _():
                        gemm1(r + 1, lax.rem(my + P - 1, P), ptmp2)

                @pl.when(ag_on)
                def _():
                    if s == 0:
                        gemm2_store(red[...], my, r - 1)
                    else:
                        gemm2_store(ag_rbuf[s - 1], lax.rem(my + s, P), r - 1)

                for h in range(2):
                    @pl.when(rs_on)
                    def _():
                        rs_rdma(s, h).wait()
                        if s < P - 2:
                            sbuf[s + 1, C(h), :] = (
                                rs_rbuf[s, C(h), :] + ptmp[C(h), :])
                            rs_rdma(s + 1, h).start()
                        else:
                            red[C(h), :] = (
                                rs_rbuf[s, C(h), :] + ptmp[C(h), :])
                            ag_rdma(0, h, red).start()

                    if s == P - 2:
                        @pl.when(r + 1 < NB)
                        def _():
                            sbuf[0, C(h), :] = ptmp2[C(h), :]
                            rs_rdma(0, h).start()

                    @pl.when(ag_on)
                    def _():
                        ag_src = red if s == 0 else ag_rbuf.at[s - 1]
                        ag_rdma(s, h, ag_src).wait()
                        if s < P - 2:
                            ag_rdma(s + 1, h, ag_rbuf.at[s]).start()

                if s == 0:
                    @pl.when(r >= 2)
                    def _():
                        gemm2_store(ag_rbuf[P - 2], lax.rem(my + P - 1, P),
                                    r - 2)
            return carry

        lax.fori_loop(0, NB + 1, tick_round, 0)

        gemm2_store(ag_rbuf[P - 2], lax.rem(my + P - 1, P), NB - 1)

    return pl.pallas_call(
        body,
        out_shape=jax.ShapeDtypeStruct((M, D), jnp.float32),
        in_specs=[
            pl.BlockSpec(memory_space=pl.ANY),
            pl.BlockSpec(memory_space=pl.ANY),
            pl.BlockSpec(memory_space=pl.ANY),
        ],
        out_specs=pl.BlockSpec(memory_space=pl.ANY),
        scratch_shapes=[
            pltpu.VMEM((D, D), jnp.bfloat16),
            pltpu.VMEM((D, D), jnp.bfloat16),
            pltpu.VMEM((R, D), jnp.float32),
            pltpu.VMEM((R, D), jnp.bfloat16),
            pltpu.VMEM((R, D), jnp.bfloat16),
            pltpu.VMEM((P - 1, R, D), jnp.bfloat16),
            pltpu.VMEM((R, D), jnp.bfloat16),
            pltpu.VMEM((P - 1, R, D), jnp.bfloat16),
            pltpu.VMEM((P - 1, R, D), jnp.bfloat16),
            pltpu.VMEM((R, D), jnp.float32),
            pltpu.SemaphoreType.DMA,
            pltpu.SemaphoreType.DMA,
            pltpu.SemaphoreType.DMA((P - 1, 2)),
            pltpu.SemaphoreType.DMA((P - 1, 2)),
            pltpu.SemaphoreType.DMA((P - 1, 2)),
            pltpu.SemaphoreType.DMA((P - 1, 2)),
        ],
        compiler_params=pltpu.CompilerParams(
            collective_id=0, vmem_limit_bytes=64 * 1024 * 1024),
    )(x, W1, W2)


# device time: 398869 ns/iter; 1.0506x vs baseline; 1.0232x over previous
import jax
import jax.numpy as jnp
from jax import lax
from jax.experimental import pallas as pl
from jax.experimental.pallas import tpu as pltpu

P = 4
D = 2048
M = 8192
NB = 8
R = 2048 // NB
BLK = 2048
HR = R // 2


def kernel(x, W1, W2):
    def body(x_hbm, w1_hbm, w2_hbm, out_hbm,
             w1b, w2b, xb, ptmp, ptmp2, sbuf, red, rs_rbuf, ag_rbuf, obuf,
             ld_sem, st_sem,
             rs_send, rs_recv, ag_send, ag_recv):
        my = lax.axis_index("i")
        left = lax.rem(my + P - 1, P)
        right = lax.rem(my + 1, P)

        def C(h):
            return pl.ds(h * HR, HR)

        def rs_rdma(slot, h):
            return pltpu.make_async_remote_copy(
                src_ref=sbuf.at[slot, C(h), :],
                dst_ref=rs_rbuf.at[slot, C(h), :],
                send_sem=rs_send.at[slot, h], recv_sem=rs_recv.at[slot, h],
                device_id=(right,), device_id_type=pl.DeviceIdType.MESH)

        def ag_rdma(slot, h, src):
            return pltpu.make_async_remote_copy(
                src_ref=src.at[C(h), :],
                dst_ref=ag_rbuf.at[slot, C(h), :],
                send_sem=ag_send.at[slot, h], recv_sem=ag_recv.at[slot, h],
                device_id=(left,), device_id_type=pl.DeviceIdType.MESH)

        barrier = pltpu.get_barrier_semaphore()
        for nbr in (left, right):
            pl.semaphore_signal(barrier, inc=1, device_id=(nbr,),
                                device_id_type=pl.DeviceIdType.MESH)
        pl.semaphore_wait(barrier, 2)

        for c in range(D // R):
            cp1 = pltpu.make_async_copy(w1_hbm.at[pl.ds(c * R, R), :], xb,
                                        ld_sem)
            cp2 = pltpu.make_async_copy(w2_hbm.at[pl.ds(c * R, R), :], obuf,
                                        st_sem)
            cp1.start()
            cp2.start()
            cp1.wait()
            w1b[pl.ds(c * R, R), :] = xb[...].astype(jnp.bfloat16)
            cp2.wait()
            w2b[pl.ds(c * R, R), :] = obuf[...].astype(jnp.bfloat16)

        def load_x(b, r):
            cp = pltpu.make_async_copy(
                x_hbm.at[pl.ds(b * BLK + r * R, R), :], xb, ld_sem)
            cp.start()
            cp.wait()

        def gemm1(r, b, dst):
            load_x(b, r)
            dst[...] = jnp.dot(xb[...].astype(jnp.bfloat16), w1b[...],
                               preferred_element_type=jnp.float32
                               ).astype(jnp.bfloat16)

        def gemm2_store(h_blk, b, q):
            obuf[...] = jnp.dot(h_blk, w2b[...],
                                preferred_element_type=jnp.float32)
            cp = pltpu.make_async_copy(
                obuf, out_hbm.at[pl.ds(b * BLK + q * R, R), :], st_sem)
            cp.start()
            cp.wait()

        load_x(lax.rem(my + P - 1, P), 0)
        sbuf[0, :, :] = jnp.dot(xb[...].astype(jnp.bfloat16), w1b[...],
                                preferred_element_type=jnp.float32
                                ).astype(jnp.bfloat16)
        for h in range(2):
            rs_rdma(0, h).start()

        def tick_round(r, carry):
            rs_on = r < NB
            ag_on = r >= 1
            for s in range(P - 1):
                @pl.when(rs_on)
                def _():
                    gemm1(r, lax.rem(my + 2 * P - s - 2, P), ptmp)

                if s == P - 2:
                    @pl.when(r + 1 < NB)
                    def _():
                        gemm1(r + 1, lax.rem(my + P - 1, P), ptmp2)

                @pl.when(ag_on)
                def _():
                    if s == 0:
                        gemm2_store(red[...], my, r - 1)
                    else:
                        gemm2_store(ag_rbuf[s - 1], lax.rem(my + s, P), r - 1)

                for h in range(2):
                    @pl.when(rs_on)
                    def _():
                        rs_rdma(s, h).wait()
                        if s < P - 2:
                            sbuf[s + 1, C(h), :] = (
                                rs_rbuf[s, C(h), :] + ptmp[C(h), :])
                            rs_rdma(s + 1, h).start()
                        else:
                            red[C(h), :] = (
                                rs_rbuf[s, C(h), :] + ptmp[C(h), :])
                            ag_rdma(0, h, red).start()

                    if s == P - 2:
                        @pl.when(r + 1 < NB)
                        def _():
                            sbuf[0, C(h), :] = ptmp2[C(h), :]
                            rs_rdma(0, h).start()

                    @pl.when(ag_on)
                    def _():
                        ag_src = red if s == 0 else ag_rbuf.at[s - 1]
                        ag_rdma(s, h, ag_src).wait()
                        if s < P - 2:
                            ag_rdma(s + 1, h, ag_rbuf.at[s]).start()

                if s == 0:
                    @pl.when(r >= 2)
                    def _():
                        gemm2_store(ag_rbuf[P - 2], lax.rem(my + P - 1, P),
                                    r - 2)
            return carry

        lax.fori_loop(0, NB + 1, tick_round, 0)

        gemm2_store(ag_rbuf[P - 2], lax.rem(my + P - 1, P), NB - 1)

    return pl.pallas_call(
        body,
        out_shape=jax.ShapeDtypeStruct((M, D), jnp.float32),
        in_specs=[
            pl.BlockSpec(memory_space=pl.ANY),
            pl.BlockSpec(memory_space=pl.ANY),
            pl.BlockSpec(memory_space=pl.ANY),
        ],
        out_specs=pl.BlockSpec(memory_space=pl.ANY),
        scratch_shapes=[
            pltpu.VMEM((D, D), jnp.bfloat16),
            pltpu.VMEM((D, D), jnp.bfloat16),
            pltpu.VMEM((R, D), jnp.float32),
            pltpu.VMEM((R, D), jnp.bfloat16),
            pltpu.VMEM((R, D), jnp.bfloat16),
            pltpu.VMEM((P - 1, R, D), jnp.bfloat16),
            pltpu.VMEM((R, D), jnp.bfloat16),
            pltpu.VMEM((P - 1, R, D), jnp.bfloat16),
            pltpu.VMEM((P - 1, R, D), jnp.bfloat16),
            pltpu.VMEM((R, D), jnp.float32),
            pltpu.SemaphoreType.DMA,
            pltpu.SemaphoreType.DMA,
            pltpu.SemaphoreType.DMA((P - 1, 2)),
            pltpu.SemaphoreType.DMA((P - 1, 2)),
            pltpu.SemaphoreType.DMA((P - 1, 2)),
            pltpu.SemaphoreType.DMA((P - 1, 2)),
        ],
        compiler_params=pltpu.CompilerParams(
            collective_id=0, vmem_limit_bytes=64 * 1024 * 1024),
    )(x, W1, W2)
